# baseline (speedup 1.0000x reference)
import functools

import jax
import jax.numpy as jnp
import numpy as np
from jax import lax
from jax.experimental import pallas as pl
from jax.experimental.pallas import tpu as pltpu
from jax.experimental.pallas import tpu_sc as plsc

B, N, M = 32, 512, 32
L = 16
LT = L * 4
NLANE = 16
GROUPS = N // NLANE

_C0 = np.float32(0.9999855460225976)
_C1 = np.float32(-0.24991878892384348)
_C2 = np.float32(0.02076114541288175)
_C3 = np.float32(-0.0006720519689284828)
_C4 = np.float32(9.532515384707557e-06)
_PI = np.float32(np.pi)
_LOG2E = np.float32(1.4426950408889634)
_RBIG = np.float32(1e18)


def _sc_body(x_hbm, nb_hbm, z_hbm, rp_hbm, out_hbm,
             xv, nbv, zv, rpv, accv, rbuf, ibuf, pbuf, sem):
    c = lax.axis_index("c")
    s = lax.axis_index("s")
    wid = s * 2 + c

    cp1 = pltpu.make_async_copy(x_hbm.at[wid], xv, sem)
    cp2 = pltpu.make_async_copy(nb_hbm.at[wid], nbv, sem)
    cp3 = pltpu.make_async_copy(z_hbm.at[wid], zv, sem)
    cp4 = pltpu.make_async_copy(rp_hbm, rpv, sem)
    cp1.start()
    cp2.start()
    cp3.start()
    cp4.start()

    zero = jnp.zeros((NLANE,), jnp.float32)

    @plsc.parallel_loop(0, N * LT // NLANE, unroll=8)
    def zbody(i):
        accv[pl.ds(i * NLANE, NLANE)] = zero

    cp1.wait()
    cp2.wait()
    cp3.wait()
    cp4.wait()

    lane = jnp.arange(NLANE, dtype=jnp.int32)

    rc_all = plsc.load_gather(rpv, [lane * 3])
    rs_all = plsc.load_gather(rpv, [lane * 3 + 1])
    e_all = plsc.load_gather(rpv, [lane * 3 + 2])
    pbuf[pl.ds(0, NLANE)] = rc_all
    pbuf[pl.ds(NLANE, NLANE)] = rs_all
    pbuf[pl.ds(2 * NLANE, NLANE)] = -e_all
    pbuf[pl.ds(3 * NLANE, NLANE)] = _PI / rc_all

    def gbody(g, carry1):
        atom = g * NLANE + lane
        a3 = atom * 3
        cx = plsc.load_gather(xv, [a3])
        cy = plsc.load_gather(xv, [a3 + 1])
        cz = plsc.load_gather(xv, [a3 + 2])
        base_off = atom * M
        acc_base = atom * LT
        gm = g * M

        @plsc.parallel_loop(0, M, unroll=4)
        def m1(m):
            off = base_off + m
            nb = plsc.load_gather(nbv, [off])
            zz = plsc.load_gather(zv, [off])
            nb3 = nb * 3
            dx = plsc.load_gather(xv, [nb3]) - cx
            dy = plsc.load_gather(xv, [nb3 + 1]) - cy
            dz = plsc.load_gather(xv, [nb3 + 2]) - cz
            r2 = dx * dx + dy * dy + dz * dz
            ii = lax.bitcast_convert_type(r2, jnp.int32)
            ii = jnp.int32(0x5F3759DF) - jnp.right_shift(ii, 1)
            y = lax.bitcast_convert_type(ii, jnp.float32)
            hr2 = np.float32(0.5) * r2
            y = y * (np.float32(1.5) - hr2 * y * y)
            y = y * (np.float32(1.5) - hr2 * y * y)
            y = y * (np.float32(1.5) - hr2 * y * y)
            r = r2 * y
            valid = (zz >= 1) & (zz <= 4)
            sl = pl.ds((gm + m) * NLANE, NLANE)
            rbuf[sl] = jnp.where(valid, r, _RBIG)
            ibuf[sl] = jnp.maximum(acc_base + (zz - 1), 0)

        return carry1

    lax.fori_loop(0, GROUPS, gbody, 0)

    def lbody(l, carry2):
        rc = plsc.load_gather(pbuf, [jnp.full((NLANE,), l, jnp.int32)])
        rs = plsc.load_gather(pbuf, [jnp.full((NLANE,), l + NLANE, jnp.int32)])
        ne = plsc.load_gather(pbuf, [jnp.full((NLANE,), l + 2 * NLANE, jnp.int32)])
        pinv = plsc.load_gather(pbuf, [jnp.full((NLANE,), l + 3 * NLANE, jnp.int32)])
        lt4 = l * 4

        @plsc.parallel_loop(0, N * M // NLANE, unroll=8)
        def m2(i):
            sl = pl.ds(i * NLANE, NLANE)
            r = rbuf[sl]
            ib = ibuf[sl]
            dd = r - rs
            kk = jnp.exp(ne * dd * dd)
            u = r * pinv
            t = u * u
            fc = _C0 + t * (_C1 + t * (_C2 + t * (_C3 + t * _C4)))
            fc = jnp.where(r <= rc, fc, np.float32(0.0))
            val = kk * fc
            plsc.addupdate_scatter(accv, [ib + lt4], val)

        return carry2

    lax.fori_loop(0, L, lbody, 0)

    pltpu.sync_copy(accv, out_hbm.at[wid])


_sc_main = functools.partial(
    pl.kernel,
    out_type=jax.ShapeDtypeStruct((B, N * LT), jnp.float32),
    mesh=plsc.VectorSubcoreMesh(core_axis_name="c", subcore_axis_name="s"),
    compiler_params=pltpu.CompilerParams(needs_layout_passes=False),
    scratch_types=[
        pltpu.VMEM((N * 3,), jnp.float32),
        pltpu.VMEM((N * M,), jnp.int32),
        pltpu.VMEM((N * M,), jnp.int32),
        pltpu.VMEM((L * 3,), jnp.float32),
        pltpu.VMEM((N * LT,), jnp.float32),
        pltpu.VMEM((N * M,), jnp.float32),
        pltpu.VMEM((N * M,), jnp.int32),
        pltpu.VMEM((4 * NLANE,), jnp.float32),
        pltpu.SemaphoreType.DMA,
    ],
)(_sc_body)


def _bn_body(x_ref, o_ref):
    x = x_ref[...]
    m = jnp.mean(x, axis=0, keepdims=True)
    d = x - m
    v = jnp.mean(d * d, axis=0, keepdims=True)
    o_ref[...] = d * lax.rsqrt(v + np.float32(0.001))


_BN_CHUNK = 2048


def _bn(layer):
    return pl.pallas_call(
        _bn_body,
        grid=(N * LT // _BN_CHUNK,),
        in_specs=[pl.BlockSpec((B, _BN_CHUNK), lambda i: (0, i))],
        out_specs=pl.BlockSpec((B, _BN_CHUNK), lambda i: (0, i)),
        out_shape=jax.ShapeDtypeStruct((B, N * LT), jnp.float32),
    )(layer)


def kernel(X, Nbrs, Nbrs_Z, radial_params):
    xf = X.reshape(B, N * 3)
    nb = Nbrs.reshape(B, N * M).astype(jnp.int32)
    zf = Nbrs_Z.reshape(B, N * M).astype(jnp.int32)
    rp = radial_params.reshape(L * 3)
    layer = _sc_main(xf, nb, zf, rp)
    out = _bn(layer)
    return out.reshape(B, N, LT)


# speedup vs baseline: 1.1135x; 1.1135x over previous
import functools

import jax
import jax.numpy as jnp
import numpy as np
from jax import lax
from jax.experimental import pallas as pl
from jax.experimental.pallas import tpu as pltpu
from jax.experimental.pallas import tpu_sc as plsc

B, N, M = 32, 512, 32
L = 16
LT = L * 4
NLANE = 16
GROUPS = N // NLANE

_C0 = np.float32(0.9999855460225976)
_C1 = np.float32(-0.24991878892384348)
_C2 = np.float32(0.02076114541288175)
_C3 = np.float32(-0.0006720519689284828)
_C4 = np.float32(9.532515384707557e-06)
_PI = np.float32(np.pi)
_LOG2E = np.float32(1.4426950408889634)
_RBIG = np.float32(1e18)


def _sc_body(x_hbm, nb_hbm, z_hbm, rp_hbm, out_hbm,
             xv, nbv, zv, rpv, accv, rbuf, ibuf, pbuf, sem):
    c = lax.axis_index("c")
    s = lax.axis_index("s")
    wid = s * 2 + c

    cp1 = pltpu.make_async_copy(x_hbm.at[wid], xv, sem)
    cp2 = pltpu.make_async_copy(nb_hbm.at[wid], nbv, sem)
    cp3 = pltpu.make_async_copy(z_hbm.at[wid], zv, sem)
    cp4 = pltpu.make_async_copy(rp_hbm, rpv, sem)
    cp1.start()
    cp2.start()
    cp3.start()
    cp4.start()

    zero = jnp.zeros((NLANE,), jnp.float32)

    @plsc.parallel_loop(0, N * LT // NLANE, unroll=8)
    def zbody(i):
        accv[pl.ds(i * NLANE, NLANE)] = zero

    cp1.wait()
    cp2.wait()
    cp3.wait()
    cp4.wait()

    lane = jnp.arange(NLANE, dtype=jnp.int32)

    rc_all = plsc.load_gather(rpv, [lane * 3])
    rs_all = plsc.load_gather(rpv, [lane * 3 + 1])
    e_all = plsc.load_gather(rpv, [lane * 3 + 2])
    pbuf[pl.ds(0, NLANE)] = rc_all
    pbuf[pl.ds(NLANE, NLANE)] = rs_all
    pbuf[pl.ds(2 * NLANE, NLANE)] = -e_all
    pbuf[pl.ds(3 * NLANE, NLANE)] = _PI / rc_all

    def gbody(g, carry1):
        atom = g * NLANE + lane
        a3 = atom * 3
        cx = plsc.load_gather(xv, [a3])
        cy = plsc.load_gather(xv, [a3 + 1])
        cz = plsc.load_gather(xv, [a3 + 2])
        base_off = atom * M
        acc_base = atom * LT
        gm = g * M

        @plsc.parallel_loop(0, M, unroll=4)
        def m1(m):
            off = base_off + m
            nb = plsc.load_gather(nbv, [off])
            zz = plsc.load_gather(zv, [off])
            nb3 = nb * 3
            dx = plsc.load_gather(xv, [nb3]) - cx
            dy = plsc.load_gather(xv, [nb3 + 1]) - cy
            dz = plsc.load_gather(xv, [nb3 + 2]) - cz
            r2 = dx * dx + dy * dy + dz * dz
            ii = lax.bitcast_convert_type(r2, jnp.int32)
            ii = jnp.int32(0x5F3759DF) - jnp.right_shift(ii, 1)
            y = lax.bitcast_convert_type(ii, jnp.float32)
            hr2 = np.float32(0.5) * r2
            y = y * (np.float32(1.5) - hr2 * y * y)
            y = y * (np.float32(1.5) - hr2 * y * y)
            y = y * (np.float32(1.5) - hr2 * y * y)
            r = r2 * y
            valid = (zz >= 1) & (zz <= 4)
            sl = pl.ds((gm + m) * NLANE, NLANE)
            rbuf[sl] = jnp.where(valid, r, _RBIG)
            ibuf[sl] = jnp.maximum(acc_base + (zz - 1), 0)

        return carry1

    lax.fori_loop(0, GROUPS, gbody, 0)

    def lbody(l, carry2):
        rc = plsc.load_gather(pbuf, [jnp.full((NLANE,), l, jnp.int32)])
        rs = plsc.load_gather(pbuf, [jnp.full((NLANE,), l + NLANE, jnp.int32)])
        ne = plsc.load_gather(pbuf, [jnp.full((NLANE,), l + 2 * NLANE, jnp.int32)])
        pinv = plsc.load_gather(pbuf, [jnp.full((NLANE,), l + 3 * NLANE, jnp.int32)])
        lt4 = l * 4

        @plsc.parallel_loop(0, N * M // NLANE, unroll=4)
        def m2(i):
            sl = pl.ds(i * NLANE, NLANE)
            r = rbuf[sl]
            ib = ibuf[sl]
            dd = r - rs
            kk = jnp.exp(ne * dd * dd)
            u = r * pinv
            t = u * u
            fc = _C0 + t * (_C1 + t * (_C2 + t * (_C3 + t * _C4)))
            fc = jnp.where(r <= rc, fc, np.float32(0.0))
            val = kk * fc
            plsc.addupdate_scatter(accv, [ib + lt4], val)

        return carry2

    lax.fori_loop(0, L, lbody, 0)

    pltpu.sync_copy(accv, out_hbm.at[wid])


_sc_main = functools.partial(
    pl.kernel,
    out_type=jax.ShapeDtypeStruct((B, N * LT), jnp.float32),
    mesh=plsc.VectorSubcoreMesh(core_axis_name="c", subcore_axis_name="s"),
    compiler_params=pltpu.CompilerParams(needs_layout_passes=False),
    scratch_types=[
        pltpu.VMEM((N * 3,), jnp.float32),
        pltpu.VMEM((N * M,), jnp.int32),
        pltpu.VMEM((N * M,), jnp.int32),
        pltpu.VMEM((L * 3,), jnp.float32),
        pltpu.VMEM((N * LT,), jnp.float32),
        pltpu.VMEM((N * M,), jnp.float32),
        pltpu.VMEM((N * M,), jnp.int32),
        pltpu.VMEM((4 * NLANE,), jnp.float32),
        pltpu.SemaphoreType.DMA,
    ],
)(_sc_body)


def _bn_body(x_ref, o_ref):
    x = x_ref[...]
    m = jnp.mean(x, axis=0, keepdims=True)
    d = x - m
    v = jnp.mean(d * d, axis=0, keepdims=True)
    o_ref[...] = d * lax.rsqrt(v + np.float32(0.001))


_BN_CHUNK = 2048


def _bn(layer):
    return pl.pallas_call(
        _bn_body,
        grid=(N * LT // _BN_CHUNK,),
        in_specs=[pl.BlockSpec((B, _BN_CHUNK), lambda i: (0, i))],
        out_specs=pl.BlockSpec((B, _BN_CHUNK), lambda i: (0, i)),
        out_shape=jax.ShapeDtypeStruct((B, N * LT), jnp.float32),
    )(layer)


def kernel(X, Nbrs, Nbrs_Z, radial_params):
    xf = X.reshape(B, N * 3)
    nb = Nbrs.reshape(B, N * M).astype(jnp.int32)
    zf = Nbrs_Z.reshape(B, N * M).astype(jnp.int32)
    rp = radial_params.reshape(L * 3)
    layer = _sc_main(xf, nb, zf, rp)
    out = _bn(layer)
    return out.reshape(B, N, LT)
